# baseline (speedup 1.0000x reference)
import functools

import jax
import jax.numpy as jnp
from jax import lax
from jax.experimental import pallas as pl
from jax.experimental.pallas import tpu as pltpu
from jax.experimental.pallas import tpu_sc as plsc

_EMB = 128
_BATCH = 512
_NWORKERS = 32
_RPW = _BATCH // _NWORKERS
_NCLS = 13
_NREL = 3
_PROWS = 4 * _RPW + 4
_NCHUNK = _EMB // 16


def _sc_stage():
    mesh = plsc.VectorSubcoreMesh(core_axis_name="c", subcore_axis_name="s")

    @functools.partial(
        pl.kernel,
        out_type=jax.ShapeDtypeStruct((_NWORKERS, _PROWS, 16), jnp.float32),
        mesh=mesh,
        scratch_types=[
            pltpu.VMEM((2, _RPW * _NCLS // 2), jnp.int32),
            pltpu.VMEM((_RPW * _NREL,), jnp.int32),
            pltpu.VMEM((_RPW * _NCLS, 2 * _EMB), jnp.float32),
            pltpu.VMEM((_RPW * _NREL, _EMB), jnp.float32),
            pltpu.VMEM((_PROWS, 16), jnp.float32),
            pltpu.SemaphoreType.DMA,
            pltpu.SemaphoreType.DMA,
            pltpu.SemaphoreType.DMA,
        ],
    )
    def sc_k(cidx_hbm, ridx_hbm, cls_hbm, rel_hbm, out_hbm,
             cidx, ridx, cbuf, rbuf, sbuf, isem, csem, rsem):
        wid = lax.axis_index("s") * 2 + lax.axis_index("c")
        nc = _RPW * _NCLS
        nr = _RPW * _NREL
        i1 = pltpu.async_copy(cidx_hbm.at[pl.ds(wid * nc, nc)], cidx, isem)
        i2 = pltpu.async_copy(ridx_hbm.at[pl.ds(wid * nr, nr)], ridx, isem)
        i1.wait()
        cg = pltpu.async_copy(cls_hbm.at[cidx], cbuf, csem)
        i2.wait()
        rg = pltpu.async_copy(rel_hbm.at[ridx], rbuf, rsem)
        cg.wait()
        rg.wait()

        zero = jnp.zeros((16,), jnp.float32)

        def halves(c, r, ch):
            cen = cbuf[r * _NCLS + c, pl.ds(ch * 16, 16)]
            off = jnp.abs(cbuf[r * _NCLS + c, pl.ds(_EMB + ch * 16, 16)])
            return cen, off

        def rvec(c, r, ch):
            return rbuf[r * _NREL + c, pl.ds(ch * 16, 16)]

        def row1(r, acc):
            for ch in range(_NCHUNK):
                cc, co = halves(0, r, ch)
                dc, do = halves(1, r, ch)
                u = jnp.maximum(jnp.abs(cc - dc) + co - do, 0.0)
                acc = acc + u * u
            return acc

        acc1 = lax.fori_loop(0, _RPW, row1, zero)

        def row2(r, _):
            sa = zero
            sb = zero
            for ch in range(_NCHUNK):
                cc, co = halves(2, r, ch)
                dc, do = halves(3, r, ch)
                ec, eo = halves(4, r, ch)
                ll = jnp.maximum(cc - co, dc - do)
                ur = jnp.minimum(cc + co, dc + do)
                dlu = ll - ur
                u = jnp.maximum(
                    jnp.abs((ll + ur) * 0.5 - ec) + jnp.abs(dlu) * 0.5 - eo, 0.0)
                sa = sa + u * u
                v = jnp.maximum(dlu, 0.0)
                sb = sb + v * v
            sbuf[r, :] = sa
            sbuf[_RPW + r, :] = sb
            return 0

        lax.fori_loop(0, _RPW, row2, 0)

        def row3(r, acc):
            for ch in range(_NCHUNK):
                cc, co = halves(5, r, ch)
                dc, do = halves(6, r, ch)
                rr = rvec(0, r, ch)
                u = jnp.maximum(jnp.abs(cc + rr - dc) + co - do, 0.0)
                acc = acc + u * u
            return acc

        acc3 = lax.fori_loop(0, _RPW, row3, zero)

        def row4(r, acc):
            for ch in range(_NCHUNK):
                cc, co = halves(7, r, ch)
                dc, do = halves(8, r, ch)
                rr = rvec(1, r, ch)
                u = jnp.maximum(jnp.abs(cc - rr - dc) + co - do, 0.0)
                acc = acc + u * u
            return acc

        acc4 = lax.fori_loop(0, _RPW, row4, zero)

        def rowd(r, _):
            sd = zero
            for ch in range(_NCHUNK):
                cc, co = halves(9, r, ch)
                dc, do = halves(10, r, ch)
                u = jnp.maximum(jnp.abs(cc - dc) - co - do, 0.0)
                sd = sd + u * u
            sbuf[2 * _RPW + r, :] = sd
            return 0

        lax.fori_loop(0, _RPW, rowd, 0)

        def rown(r, _):
            sn = zero
            for ch in range(_NCHUNK):
                cc, co = halves(11, r, ch)
                dc, do = halves(12, r, ch)
                rr = rvec(2, r, ch)
                u = jnp.maximum(jnp.abs(cc + rr - dc) - co - do, 0.0)
                sn = sn + u * u
            sbuf[3 * _RPW + r, :] = sn
            return 0

        lax.fori_loop(0, _RPW, rown, 0)

        sbuf[4 * _RPW, :] = acc1
        sbuf[4 * _RPW + 1, :] = acc3
        sbuf[4 * _RPW + 2, :] = acc4
        sbuf[4 * _RPW + 3, :] = zero
        pltpu.sync_copy(sbuf, out_hbm.at[wid])

    return sc_k


def _combine_body(x_ref, o_ref):
    x = x_ref[...]
    inv_b = 1.0 / _BATCH
    sa = jnp.sum(x[:, 0:_RPW, :], axis=2)
    sb = jnp.sum(x[:, _RPW:2 * _RPW, :], axis=2)
    sd = jnp.sum(x[:, 2 * _RPW:3 * _RPW, :], axis=2)
    sn = jnp.sum(x[:, 3 * _RPW:4 * _RPW, :], axis=2)
    p0 = jnp.sum(x[:, 4 * _RPW, :])
    p5 = jnp.sum(x[:, 4 * _RPW + 1, :])
    p6 = jnp.sum(x[:, 4 * _RPW + 2, :])
    a = jnp.sqrt(sa)
    b = jnp.sqrt(sb)
    p1 = jnp.sum(a)
    p2 = jnp.sum(sa)
    p3 = jnp.sum(b)
    p4 = jnp.sum(sb)
    p7 = jnp.sum(jnp.maximum(2.0 - jnp.sqrt(sd), 0.0) ** 2)
    p8 = jnp.sum(jnp.sqrt(sn))
    p9 = jnp.sum(sn)
    loss = (p0 * inv_b
            + p2 * inv_b + 2.0 * (p1 * inv_b) * (p3 * inv_b) + p4 * inv_b
            + p7 * inv_b
            + p5 * inv_b + p6 * inv_b
            + 4.0 - 4.0 * p8 * inv_b + p9 * inv_b)
    o_ref[0, 0] = loss


def _tc_combine(partials):
    return pl.pallas_call(
        _combine_body,
        out_shape=jax.ShapeDtypeStruct((1, 1), jnp.float32),
        in_specs=[pl.BlockSpec(memory_space=pltpu.VMEM)],
        out_specs=pl.BlockSpec(memory_space=pltpu.SMEM),
    )(partials)


def kernel(nf1, nf2, nf3, nf4, disjoint, nf3_neg, class_emb, rel_emb):
    b = _BATCH
    cls_idx = jnp.concatenate(
        [nf1[:b], nf2[:b], nf3[:b, 0:1], nf3[:b, 2:3], nf4[:b, 1:3],
         disjoint[:b], nf3_neg[:b, 0:1], nf3_neg[:b, 2:3]], axis=1)
    rel_idx = jnp.concatenate(
        [nf3[:b, 1:2], nf4[:b, 0:1], nf3_neg[:b, 1:2]], axis=1)
    partials = _sc_stage()(
        cls_idx.reshape(_NWORKERS, 2, _RPW * _NCLS // 2),
        rel_idx.reshape(-1), class_emb, rel_emb)
    return _tc_combine(partials)[0, 0]


# speedup vs baseline: 1.0576x; 1.0576x over previous
import functools

import jax
import jax.numpy as jnp
from jax import lax
from jax.experimental import pallas as pl
from jax.experimental.pallas import tpu as pltpu
from jax.experimental.pallas import tpu_sc as plsc

_EMB = 128
_BATCH = 512
_NWORKERS = 32
_RPW = _BATCH // _NWORKERS
_NCLS = 13
_NREL = 3
_PROWS = 4 * _RPW + 4
_NCHUNK = _EMB // 16
_UNROLL = 1


def _sc_stage():
    mesh = plsc.VectorSubcoreMesh(core_axis_name="c", subcore_axis_name="s")

    @functools.partial(
        pl.kernel,
        out_type=jax.ShapeDtypeStruct((_NWORKERS, _PROWS, 16), jnp.float32),
        mesh=mesh,
        scratch_types=[
            pltpu.VMEM((16, _RPW), jnp.int32),
            pltpu.VMEM((_NCLS, _RPW, 2 * _EMB), jnp.float32),
            pltpu.VMEM((_NREL, _RPW, _EMB), jnp.float32),
            pltpu.VMEM((_PROWS, 16), jnp.float32),
            [pltpu.SemaphoreType.DMA] * 6,
        ],
    )
    def sc_k(idx_hbm, cls_hbm, rel_hbm, out_hbm, idxv, cbuf, rbuf, sbuf, sems):
        wid = lax.axis_index("s") * 2 + lax.axis_index("c")
        pltpu.sync_copy(idx_hbm.at[wid], idxv)

        def cgather(t, dst, g):
            return pltpu.async_copy(cls_hbm.at[idxv[t, :]], cbuf.at[dst], sems[g])

        def rgather(t, dst, g):
            return pltpu.async_copy(rel_hbm.at[idxv[t, :]], rbuf.at[dst], sems[g])

        groups = [
            [cgather(0, 0, 0), cgather(1, 1, 0)],
            [cgather(2, 2, 1), cgather(3, 3, 1), cgather(4, 4, 1)],
            [cgather(5, 5, 2), cgather(7, 6, 2), rgather(6, 0, 2)],
            [cgather(9, 7, 3), cgather(10, 8, 3), rgather(8, 1, 3)],
            [cgather(11, 9, 4), cgather(12, 10, 4)],
            [cgather(13, 11, 5), cgather(15, 12, 5), rgather(14, 2, 5)],
        ]

        zero = jnp.zeros((16,), jnp.float32)

        def halves(t, r, ch):
            c = cbuf[t, r, pl.ds(ch * 16, 16)]
            o = jnp.abs(cbuf[t, r, pl.ds(_EMB + ch * 16, 16)])
            return c, o

        for d in groups[0]:
            d.wait()

        def row1(i, acc):
            r0 = i * _UNROLL
            for k in range(_UNROLL):
                r = r0 + k
                for ch in range(_NCHUNK):
                    cc, co = halves(0, r, ch)
                    dc, do = halves(1, r, ch)
                    u = jnp.maximum(jnp.abs(cc - dc) + co - do, 0.0)
                    acc = acc + u * u
            return acc

        acc1 = lax.fori_loop(0, _RPW // _UNROLL, row1, zero)

        for d in groups[1]:
            d.wait()

        def row2(i, _):
            r0 = i * _UNROLL
            for k in range(_UNROLL):
                r = r0 + k
                sa = zero
                sb = zero
                for ch in range(_NCHUNK):
                    cc, co = halves(2, r, ch)
                    dc, do = halves(3, r, ch)
                    ec, eo = halves(4, r, ch)
                    ll = jnp.maximum(cc - co, dc - do)
                    ur = jnp.minimum(cc + co, dc + do)
                    dlu = ll - ur
                    u = jnp.maximum(
                        jnp.abs((ll + ur) * 0.5 - ec) + jnp.abs(dlu) * 0.5 - eo,
                        0.0)
                    sa = sa + u * u
                    v = jnp.maximum(dlu, 0.0)
                    sb = sb + v * v
                sbuf[r, :] = sa
                sbuf[_RPW + r, :] = sb
            return 0

        lax.fori_loop(0, _RPW // _UNROLL, row2, 0)

        for d in groups[2]:
            d.wait()

        def row3(i, acc):
            r0 = i * _UNROLL
            for k in range(_UNROLL):
                r = r0 + k
                for ch in range(_NCHUNK):
                    cc, co = halves(5, r, ch)
                    dc, do = halves(6, r, ch)
                    rr = rbuf[0, r, pl.ds(ch * 16, 16)]
                    u = jnp.maximum(jnp.abs(cc + rr - dc) + co - do, 0.0)
                    acc = acc + u * u
            return acc

        acc3 = lax.fori_loop(0, _RPW // _UNROLL, row3, zero)

        for d in groups[3]:
            d.wait()

        def row4(i, acc):
            r0 = i * _UNROLL
            for k in range(_UNROLL):
                r = r0 + k
                for ch in range(_NCHUNK):
                    cc, co = halves(7, r, ch)
                    dc, do = halves(8, r, ch)
                    rr = rbuf[1, r, pl.ds(ch * 16, 16)]
                    u = jnp.maximum(jnp.abs(cc - rr - dc) + co - do, 0.0)
                    acc = acc + u * u
            return acc

        acc4 = lax.fori_loop(0, _RPW // _UNROLL, row4, zero)

        for d in groups[4]:
            d.wait()

        def rowd(i, _):
            r0 = i * _UNROLL
            for k in range(_UNROLL):
                r = r0 + k
                sd = zero
                for ch in range(_NCHUNK):
                    cc, co = halves(9, r, ch)
                    dc, do = halves(10, r, ch)
                    u = jnp.maximum(jnp.abs(cc - dc) - co - do, 0.0)
                    sd = sd + u * u
                sbuf[2 * _RPW + r, :] = sd
            return 0

        lax.fori_loop(0, _RPW // _UNROLL, rowd, 0)

        for d in groups[5]:
            d.wait()

        def rown(i, _):
            r0 = i * _UNROLL
            for k in range(_UNROLL):
                r = r0 + k
                sn = zero
                for ch in range(_NCHUNK):
                    cc, co = halves(11, r, ch)
                    dc, do = halves(12, r, ch)
                    rr = rbuf[2, r, pl.ds(ch * 16, 16)]
                    u = jnp.maximum(jnp.abs(cc + rr - dc) - co - do, 0.0)
                    sn = sn + u * u
                sbuf[3 * _RPW + r, :] = sn
            return 0

        lax.fori_loop(0, _RPW // _UNROLL, rown, 0)

        sbuf[4 * _RPW, :] = acc1
        sbuf[4 * _RPW + 1, :] = acc3
        sbuf[4 * _RPW + 2, :] = acc4
        sbuf[4 * _RPW + 3, :] = zero
        pltpu.sync_copy(sbuf, out_hbm.at[wid])

    return sc_k


def _combine_body(x_ref, o_ref):
    x = x_ref[...]
    inv_b = 1.0 / _BATCH
    sa = jnp.sum(x[:, 0:_RPW, :], axis=2)
    sb = jnp.sum(x[:, _RPW:2 * _RPW, :], axis=2)
    sd = jnp.sum(x[:, 2 * _RPW:3 * _RPW, :], axis=2)
    sn = jnp.sum(x[:, 3 * _RPW:4 * _RPW, :], axis=2)
    p0 = jnp.sum(x[:, 4 * _RPW, :])
    p5 = jnp.sum(x[:, 4 * _RPW + 1, :])
    p6 = jnp.sum(x[:, 4 * _RPW + 2, :])
    a = jnp.sqrt(sa)
    b = jnp.sqrt(sb)
    p1 = jnp.sum(a)
    p2 = jnp.sum(sa)
    p3 = jnp.sum(b)
    p4 = jnp.sum(sb)
    p7 = jnp.sum(jnp.maximum(2.0 - jnp.sqrt(sd), 0.0) ** 2)
    p8 = jnp.sum(jnp.sqrt(sn))
    p9 = jnp.sum(sn)
    loss = (p0 * inv_b
            + p2 * inv_b + 2.0 * (p1 * inv_b) * (p3 * inv_b) + p4 * inv_b
            + p7 * inv_b
            + p5 * inv_b + p6 * inv_b
            + 4.0 - 4.0 * p8 * inv_b + p9 * inv_b)
    o_ref[0, 0] = loss


def _tc_combine(partials):
    return pl.pallas_call(
        _combine_body,
        out_shape=jax.ShapeDtypeStruct((1, 1), jnp.float32),
        in_specs=[pl.BlockSpec(memory_space=pltpu.VMEM)],
        out_specs=pl.BlockSpec(memory_space=pltpu.SMEM),
    )(partials)


def kernel(nf1, nf2, nf3, nf4, disjoint, nf3_neg, class_emb, rel_emb):
    b = _BATCH
    cols = jnp.concatenate(
        [nf1[:b], nf2[:b], nf3[:b], nf4[:b], disjoint[:b], nf3_neg[:b]], axis=1)
    idx_w = cols.reshape(_NWORKERS, _RPW, 16).transpose(0, 2, 1)
    partials = _sc_stage()(idx_w, class_emb, rel_emb)
    return _tc_combine(partials)[0, 0]
